# baseline (speedup 1.0000x reference)
import functools

import jax
import jax.numpy as jnp
from jax import lax
from jax.experimental import pallas as pl
from jax.experimental.pallas import tpu as pltpu
from jax.experimental.pallas import tpu_sc as plsc

BATCH = 8192
F = 4096
L = 16

NUM_CORES = 2
NUM_SUBCORES = 16
NW = NUM_CORES * NUM_SUBCORES
ROWS_PER_W = BATCH // NW
R = 8
NCHUNK = ROWS_PER_W // R
FQ = F // 4

_mesh = plsc.VectorSubcoreMesh(core_axis_name="c", subcore_axis_name="s")


@functools.partial(
    pl.kernel,
    out_type=jax.ShapeDtypeStruct((BATCH, F), jnp.float32),
    mesh=_mesh,
    scratch_types=[
        pltpu.VMEM((F,), jnp.int32),
        pltpu.VMEM((R, F), jnp.float32),
        pltpu.VMEM((R, F), jnp.float32),
        pltpu.VMEM((R, F), jnp.float32),
        pltpu.VMEM((R, FQ), jnp.float32),
        pltpu.VMEM((R, FQ), jnp.float32),
        pltpu.SemaphoreType.DMA,
        pltpu.SemaphoreType.DMA,
        pltpu.SemaphoreType.DMA,
        pltpu.SemaphoreType.DMA,
        pltpu.SemaphoreType.DMA,
    ],
    compiler_params=pltpu.CompilerParams(
        needs_layout_passes=False,
        use_tc_tiling_on_sc=True,
    ),
)
def _shuffle(x_hbm, perm_hbm, out_hbm, perm_v, in0, in1, in2, out0, out1,
             isem0, isem1, isem2, osem0, osem1):
    wid = lax.axis_index("s") * NUM_CORES + lax.axis_index("c")
    base = wid * ROWS_PER_W

    pltpu.sync_copy(perm_hbm, perm_v)

    ins = (in0, in1, in2)
    outs = (out0, out1)
    isems = (isem0, isem1, isem2)
    osems = (osem0, osem1)

    def src(c):
        return x_hbm.at[pl.ds(base + c * R, R), :]

    def dst(c, q):
        return out_hbm.at[pl.ds(base + c * R, R), pl.ds(q * FQ, FQ)]

    def start_in(c, b):
        pltpu.async_copy(src(c), ins[b], isems[b])

    def wait_in(c, b):
        pltpu.make_async_copy(src(c), ins[b], isems[b]).wait()

    def start_out(c, q, b):
        pltpu.async_copy(outs[b], dst(c, q), osems[b])

    def wait_out(c, q, b):
        pltpu.make_async_copy(outs[b], dst(c, q), osems[b]).wait()

    row_ids = [jnp.full((L,), r, dtype=jnp.int32) for r in range(R)]

    def gather(bi, q, ob):
        iv = ins[bi]
        ov = outs[ob]

        @plsc.parallel_loop(0, FQ // L, unroll=2)
        def body(i):
            idx = perm_v[pl.ds((q * (FQ // L) + i) * L, L)]
            vals = [plsc.load_gather(iv, [row_ids[r], idx]) for r in range(R)]
            for r in range(R):
                ov[r, pl.ds(i * L, L)] = vals[r]

    def process(c, b, first=False, prefetch=True):
        wait_in(c, b)
        for q in range(4):
            ob = q % 2
            if not (first and q < 2):
                if q >= 2:
                    wait_out(c, q - 2, ob)
                else:
                    wait_out(c - 1, q + 2, ob)
            gather(b, q, ob)
            start_out(c, q, ob)
        if prefetch:
            start_in(c + 3, b)

    start_in(0, 0)
    start_in(1, 1)
    start_in(2, 2)
    process(0, 0, first=True)

    def triple(p, carry):
        process(p * 3 + 1, 1)
        process(p * 3 + 2, 2)
        process(p * 3 + 3, 0)
        return carry

    lax.fori_loop(0, (NCHUNK - 5) // 3, triple, 0)

    process(NCHUNK - 4, 1)
    process(NCHUNK - 3, 2, prefetch=False)
    process(NCHUNK - 2, 0, prefetch=False)
    process(NCHUNK - 1, 1, prefetch=False)
    wait_out(NCHUNK - 1, 2, 0)
    wait_out(NCHUNK - 1, 3, 1)


def kernel(x, perm):
    perm32 = perm.astype(jnp.int32)
    return _shuffle(x, perm32)


# speedup vs baseline: 1.0147x; 1.0147x over previous
import functools

import jax
import jax.numpy as jnp
from jax import lax
from jax.experimental import pallas as pl
from jax.experimental.pallas import tpu as pltpu
from jax.experimental.pallas import tpu_sc as plsc

BATCH = 8192
F = 4096
L = 16

NUM_CORES = 2
NUM_SUBCORES = 16
NW = NUM_CORES * NUM_SUBCORES
ROWS_PER_W = BATCH // NW
R = 8
NCHUNK = ROWS_PER_W // R
FH = F // 2

_mesh = plsc.VectorSubcoreMesh(core_axis_name="c", subcore_axis_name="s")


@functools.partial(
    pl.kernel,
    out_type=jax.ShapeDtypeStruct((BATCH, F), jnp.float32),
    mesh=_mesh,
    scratch_types=[
        pltpu.VMEM((F,), jnp.int32),
        pltpu.VMEM((R, F), jnp.float32),
        pltpu.VMEM((R, F), jnp.float32),
        pltpu.VMEM((R, FH), jnp.float32),
        pltpu.VMEM((R, FH), jnp.float32),
        pltpu.VMEM((R, FH), jnp.float32),
        pltpu.SemaphoreType.DMA,
        pltpu.SemaphoreType.DMA,
        pltpu.SemaphoreType.DMA,
        pltpu.SemaphoreType.DMA,
        pltpu.SemaphoreType.DMA,
    ],
    compiler_params=pltpu.CompilerParams(
        needs_layout_passes=False,
        use_tc_tiling_on_sc=True,
    ),
)
def _shuffle(x_hbm, perm_hbm, out_hbm, perm_v, in0, in1, out0, out1, out2,
             isem0, isem1, osem0, osem1, osem2):
    wid = lax.axis_index("s") * NUM_CORES + lax.axis_index("c")
    base = wid * ROWS_PER_W

    pltpu.sync_copy(perm_hbm, perm_v)

    ins = (in0, in1)
    outs = (out0, out1, out2)
    isems = (isem0, isem1)
    osems = (osem0, osem1, osem2)

    def src(c):
        return x_hbm.at[pl.ds(base + c * R, R), :]

    def dst(c, h):
        return out_hbm.at[pl.ds(base + c * R, R), pl.ds(h * FH, FH)]

    def start_in(c, b):
        pltpu.async_copy(src(c), ins[b], isems[b])

    def wait_in(c, b):
        pltpu.make_async_copy(src(c), ins[b], isems[b]).wait()

    def start_out(c, h, b):
        pltpu.async_copy(outs[b], dst(c, h), osems[b])

    def wait_out(c, h, b):
        pltpu.make_async_copy(outs[b], dst(c, h), osems[b]).wait()

    row_ids = [jnp.full((L,), r, dtype=jnp.int32) for r in range(R)]

    def gather(bi, h, ob):
        iv = ins[bi]
        ov = outs[ob]

        @plsc.parallel_loop(0, FH // L, unroll=2)
        def body(i):
            idx = perm_v[pl.ds((h * (FH // L) + i) * L, L)]
            vals = [plsc.load_gather(iv, [row_ids[r], idx]) for r in range(R)]
            for r in range(R):
                ov[r, pl.ds(i * L, L)] = vals[r]

    def proc(c, ib, ob0, ob1, w0=True, w1=True, pf=True):
        wait_in(c, ib)
        if w0:
            wait_out(c - 2, 1, ob0)
        gather(ib, 0, ob0)
        start_out(c, 0, ob0)
        if w1:
            wait_out(c - 1, 0, ob1)
        gather(ib, 1, ob1)
        start_out(c, 1, ob1)
        if pf:
            start_in(c + 2, ib)

    start_in(0, 0)
    start_in(1, 1)
    proc(0, 0, 0, 1, w0=False, w1=False)
    proc(1, 1, 2, 0, w0=False)
    proc(2, 0, 1, 2)

    def sext(p, carry):
        c = p * 6 + 3
        proc(c + 0, 1, 0, 1)
        proc(c + 1, 0, 2, 0)
        proc(c + 2, 1, 1, 2)
        proc(c + 3, 0, 0, 1)
        proc(c + 4, 1, 2, 0)
        proc(c + 5, 0, 1, 2)
        return carry

    lax.fori_loop(0, 4, sext, 0)

    proc(27, 1, 0, 1)
    proc(28, 0, 2, 0)
    proc(29, 1, 1, 2)
    proc(30, 0, 0, 1, pf=False)
    proc(31, 1, 2, 0, pf=False)
    wait_out(30, 1, 1)
    wait_out(31, 0, 2)
    wait_out(31, 1, 0)


def kernel(x, perm):
    perm32 = perm.astype(jnp.int32)
    return _shuffle(x, perm32)


# speedup vs baseline: 1.0399x; 1.0248x over previous
import functools

import jax
import jax.numpy as jnp
from jax import lax
from jax.experimental import pallas as pl
from jax.experimental.pallas import tpu as pltpu
from jax.experimental.pallas import tpu_sc as plsc

BATCH = 8192
F = 4096
L = 16

NUM_CORES = 2
NUM_SUBCORES = 16
NW = NUM_CORES * NUM_SUBCORES
ROWS_PER_W = BATCH // NW
R = 8
NCHUNK = ROWS_PER_W // R
FH = F // 2

_mesh = plsc.VectorSubcoreMesh(core_axis_name="c", subcore_axis_name="s")


@functools.partial(
    pl.kernel,
    out_type=jax.ShapeDtypeStruct((BATCH, F), jnp.float32),
    mesh=_mesh,
    scratch_types=[
        pltpu.VMEM((F,), jnp.int32),
        pltpu.VMEM((R, F), jnp.float32),
        pltpu.VMEM((R, F), jnp.float32),
        pltpu.VMEM((R, FH), jnp.float32),
        pltpu.VMEM((R, FH), jnp.float32),
        pltpu.SemaphoreType.DMA,
        pltpu.SemaphoreType.DMA,
        pltpu.SemaphoreType.DMA,
        pltpu.SemaphoreType.DMA,
    ],
    compiler_params=pltpu.CompilerParams(
        needs_layout_passes=False,
        use_tc_tiling_on_sc=True,
    ),
)
def _shuffle(x_hbm, perm_hbm, out_hbm, perm_v, in0, in1, out0, out1,
             isem0, isem1, osem0, osem1):
    wid = lax.axis_index("s") * NUM_CORES + lax.axis_index("c")
    base = wid * ROWS_PER_W

    pltpu.sync_copy(perm_hbm, perm_v)

    ins = (in0, in1)
    outs = (out0, out1)
    isems = (isem0, isem1)
    osems = (osem0, osem1)

    def src(c):
        return x_hbm.at[pl.ds(base + c * R, R), :]

    def dst(c, h):
        return out_hbm.at[pl.ds(base + c * R, R), pl.ds(h * FH, FH)]

    def start_in(c, b):
        pltpu.async_copy(src(c), ins[b], isems[b])

    def wait_in(c, b):
        pltpu.make_async_copy(src(c), ins[b], isems[b]).wait()

    def start_out(c, h, b):
        pltpu.async_copy(outs[b], dst(c, h), osems[b])

    def wait_out(c, h, b):
        pltpu.make_async_copy(outs[b], dst(c, h), osems[b]).wait()

    row_ids = [jnp.full((L,), r, dtype=jnp.int32) for r in range(R)]

    def gather(bi, h):
        iv = ins[bi]
        ov = outs[h]

        @plsc.parallel_loop(0, FH // L, unroll=2)
        def body(i):
            idx = perm_v[pl.ds((h * (FH // L) + i) * L, L)]
            vals = [plsc.load_gather(iv, [row_ids[r], idx]) for r in range(R)]
            for r in range(R):
                ov[r, pl.ds(i * L, L)] = vals[r]

    def process_steady(c, b):
        wait_in(c, b)
        for h in range(2):
            wait_out(c - 1, h, h)
            gather(b, h)
            start_out(c, h, h)
        start_in(c + 2, b)

    start_in(0, 0)
    start_in(1, 1)
    wait_in(0, 0)
    for h in range(2):
        gather(0, h)
        start_out(0, h, h)
    start_in(2, 0)

    def pair(p, carry):
        process_steady(p * 2 + 1, 1)
        process_steady(p * 2 + 2, 0)
        return carry

    lax.fori_loop(0, NCHUNK // 2 - 2, pair, 0)

    process_steady(NCHUNK - 3, 1)
    c = NCHUNK - 2
    wait_in(c, 0)
    for h in range(2):
        wait_out(c - 1, h, h)
        gather(0, h)
        start_out(c, h, h)
    c = NCHUNK - 1
    wait_in(c, 1)
    for h in range(2):
        wait_out(c - 1, h, h)
        gather(1, h)
        start_out(c, h, h)
    for h in range(2):
        wait_out(NCHUNK - 1, h, h)


def kernel(x, perm):
    perm32 = perm.astype(jnp.int32)
    return _shuffle(x, perm32)
